# baseline (speedup 1.0000x reference)
import functools

import jax
import jax.numpy as jnp
from jax import lax
from jax.experimental import pallas as pl
from jax.experimental.pallas import tpu as pltpu
from jax.experimental.pallas import tpu_sc as plsc

_B = 16384
_D = 64
_NC = 2
_NS = 16
_L = 16
_NW = _NC * _NS
_RPW = _B // _NW
_NCHUNK = 4
_CHUNK = _RPW // _NCHUNK


def _sc_partials(users, items, scores, sample_weight, user_table, item_table):
    mesh = plsc.VectorSubcoreMesh(core_axis_name="c", subcore_axis_name="s")

    @functools.partial(
        pl.kernel,
        mesh=mesh,
        out_type=jax.ShapeDtypeStruct((_NW, _L), jnp.float32),
        compiler_params=pltpu.CompilerParams(
            needs_layout_passes=False, use_tc_tiling_on_sc=False),
        scratch_types=[
            pltpu.VMEM((_NCHUNK, _CHUNK), jnp.int32),
            pltpu.VMEM((_NCHUNK, _CHUNK), jnp.int32),
            pltpu.VMEM((_RPW,), jnp.float32),
            pltpu.VMEM((_RPW,), jnp.float32),
            pltpu.VMEM((_RPW, _D), jnp.float32),
            pltpu.VMEM((_RPW, _D), jnp.float32),
            pltpu.VMEM((_L,), jnp.float32),
            pltpu.SemaphoreType.DMA,
        ],
    )
    def k(users_h, items_h, scores_h, w_h, ut_h, it_h, out_h,
          uidx, iidx, sc_v, w_v, urows, irows, ostage, sem):
        wid = lax.axis_index("s") * _NC + lax.axis_index("c")
        base = wid * _RPW

        for j in range(_NCHUNK):
            off = base + j * _CHUNK
            pltpu.sync_copy(users_h.at[pl.ds(off, _CHUNK)], uidx.at[j])
            pltpu.sync_copy(items_h.at[pl.ds(off, _CHUNK)], iidx.at[j])
        pltpu.sync_copy(scores_h.at[pl.ds(base, _RPW)], sc_v)
        pltpu.sync_copy(w_h.at[pl.ds(base, _RPW)], w_v)

        copies = []
        for j in range(_NCHUNK):
            dst = pl.ds(j * _CHUNK, _CHUNK)
            copies.append(pltpu.async_copy(ut_h.at[uidx.at[j]], urows.at[dst], sem))
            copies.append(pltpu.async_copy(it_h.at[iidx.at[j]], irows.at[dst], sem))
        for c in copies:
            c.wait()

        def group(g, acc):
            r0 = g * _L
            svec = sc_v[pl.ds(r0, _L)]
            wvec = w_v[pl.ds(r0, _L)]
            for j in range(_L):
                r = r0 + j
                p = urows[r, pl.ds(0, _L)] * irows[r, pl.ds(0, _L)]
                for c in range(1, _D // _L):
                    p = p + urows[r, pl.ds(c * _L, _L)] * irows[r, pl.ds(c * _L, _L)]
                pred = jnp.sum(p)
                e = pred - svec[j]
                acc = acc + e * e * wvec[j]
            return acc

        total = lax.fori_loop(0, _RPW // _L, group, jnp.float32(0.0))
        ostage[...] = jnp.where(lax.iota(jnp.int32, _L) == 0, total, 0.0)
        pltpu.sync_copy(ostage, out_h.at[wid])

    return k(users, items, scores, sample_weight, user_table, item_table)


def _finalize(partials):
    def body(p_ref, o_ref):
        o_ref[0, 0] = jnp.sum(p_ref[...]) * (1.0 / _B)

    out = pl.pallas_call(
        body,
        out_shape=jax.ShapeDtypeStruct((1, 1), jnp.float32),
        out_specs=pl.BlockSpec(memory_space=pltpu.SMEM),
    )(partials)
    return out[0, 0]


def kernel(users, items, scores, sample_weight, user_table, item_table):
    partials = _sc_partials(users, items, scores, sample_weight,
                            user_table, item_table)
    return _finalize(partials)
